# Initial kernel scaffold; baseline (speedup 1.0000x reference)
import functools

import jax
import jax.numpy as jnp
from jax import lax
from jax.experimental import pallas as pl
from jax.experimental.pallas import tpu as pltpu
from jax.experimental.pallas import tpu_sc as plsc

_T = 1000
_TPAD = 1024
_B = 16384
_NW = 32
_BPW = _B // _NW


def _sqrt16(x):
    bits = plsc.bitcast(x, jnp.int32)
    y = plsc.bitcast((bits >> 1) + jnp.int32(0x1FBD1DF5), jnp.float32)
    for _ in range(3):
        y = 0.5 * (y + x / y)
    return y


@functools.partial(
    pl.kernel,
    mesh=plsc.VectorSubcoreMesh(core_axis_name="c", subcore_axis_name="s"),
    out_type=jax.ShapeDtypeStruct((2 * _B,), jnp.float32),
    scratch_types=[
        pltpu.VMEM((_BPW,), jnp.int32),
        pltpu.VMEM((_TPAD,), jnp.float32),
        pltpu.VMEM((_TPAD,), jnp.float32),
        pltpu.VMEM((_TPAD,), jnp.float32),
        pltpu.VMEM((2 * _BPW,), jnp.float32),
    ],
)
def _sc_encode(t_hbm, tab_hbm, out_hbm, idx_v, tab_v, sa_v, sb_v, out_v):
    wid = lax.axis_index("s") * 2 + lax.axis_index("c")
    base = wid * _BPW
    pltpu.sync_copy(tab_hbm, tab_v)
    pltpu.sync_copy(t_hbm.at[pl.ds(base, _BPW)], idx_v)

    def tbody(i, c):
        x = tab_v[pl.ds(i * 16, 16)]
        sa_v[pl.ds(i * 16, 16)] = _sqrt16(x)
        sb_v[pl.ds(i * 16, 16)] = _sqrt16(1.0 - x)
        return c

    lax.fori_loop(0, _TPAD // 16, tbody, 0)

    lane = lax.iota(jnp.int32, 16)

    def gbody(j, c):
        tv = idx_v[pl.ds(j * 16, 16)]
        a = plsc.load_gather(sa_v, [tv])
        b = plsc.load_gather(sb_v, [tv])
        k = j * 32 + 2 * lane
        plsc.store_scatter(out_v, [k], a)
        plsc.store_scatter(out_v, [k + 1], b)
        return c

    lax.fori_loop(0, _BPW // 16, gbody, 0)

    pltpu.sync_copy(out_v, out_hbm.at[pl.ds(2 * base, 2 * _BPW)])


def kernel(t, alphas_cumprod):
    tab = jnp.pad(alphas_cumprod, (0, _TPAD - _T), constant_values=0.5)
    flat = _sc_encode(t.astype(jnp.int32), tab)
    return flat.reshape(_B, 2)


# speedup vs baseline: 2.5489x; 2.5489x over previous
import functools

import jax
import jax.numpy as jnp
from jax import lax
from jax.experimental import pallas as pl
from jax.experimental.pallas import tpu as pltpu
from jax.experimental.pallas import tpu_sc as plsc

_T = 1000
_TPAD = 1024
_B = 16384
_NW = 32
_BPW = _B // _NW


def _sqrt16(x):
    bits = lax.bitcast_convert_type(x, jnp.int32)
    y = lax.bitcast_convert_type((bits >> 1) + jnp.int32(0x1FBD1DF5),
                                 jnp.float32)
    for _ in range(3):
        y = 0.5 * (y + x / y)
    return y


@functools.partial(
    pl.kernel,
    mesh=plsc.VectorSubcoreMesh(core_axis_name="c", subcore_axis_name="s"),
    out_type=jax.ShapeDtypeStruct((2 * _B,), jnp.float32),
    scratch_types=[
        pltpu.VMEM((_BPW,), jnp.int32),
        pltpu.VMEM((_TPAD,), jnp.float32),
        pltpu.VMEM((_TPAD,), jnp.float32),
        pltpu.VMEM((_TPAD,), jnp.float32),
        pltpu.VMEM((2 * _BPW,), jnp.float32),
    ],
    compiler_params=pltpu.CompilerParams(needs_layout_passes=False),
)
def _sc_encode(t_hbm, tab_hbm, out_hbm, idx_v, tab_v, sa_v, sb_v, out_v):
    wid = lax.axis_index("s") * 2 + lax.axis_index("c")
    base = wid * _BPW
    pltpu.sync_copy(tab_hbm, tab_v)
    pltpu.sync_copy(t_hbm.at[pl.ds(base, _BPW)], idx_v)

    def tbody(i, c):
        x = tab_v[pl.ds(i * 16, 16)]
        sa_v[pl.ds(i * 16, 16)] = _sqrt16(x)
        sb_v[pl.ds(i * 16, 16)] = _sqrt16(1.0 - x)
        return c

    lax.fori_loop(0, _TPAD // 16, tbody, 0)

    lane = lax.iota(jnp.int32, 16)

    def gbody(j, c):
        tv = idx_v[pl.ds(j * 16, 16)]
        a = plsc.load_gather(sa_v, [tv])
        b = plsc.load_gather(sb_v, [tv])
        k = j * 32 + 2 * lane
        plsc.store_scatter(out_v, [k], a)
        plsc.store_scatter(out_v, [k + 1], b)
        return c

    lax.fori_loop(0, _BPW // 16, gbody, 0)

    pltpu.sync_copy(out_v, out_hbm.at[pl.ds(2 * base, 2 * _BPW)])


def kernel(t, alphas_cumprod):
    tab = jnp.pad(alphas_cumprod, (0, _TPAD - _T), constant_values=0.5)
    flat = _sc_encode(t.astype(jnp.int32), tab)
    return flat.reshape(_B, 2)
